# baseline (speedup 1.0000x reference)
import functools

import jax
import jax.numpy as jnp
from jax import lax
from jax.experimental import pallas as pl
from jax.experimental.pallas import tpu as pltpu
from jax.experimental.pallas import tpu_sc as plsc

N_NODES = 100000
N_EDGES = 6400000

NC = 2
NS = 16
NW = NC * NS
EW = N_EDGES // NW
G = 80
CHUNK = 1600
NG = CHUNK // G
NCHUNK = EW // CHUNK
NB = CHUNK // 16
PT = 6256
N_PAD = PT * NS
DRN = PT // 4


def _sc_scatter(disp_flat, dst_idx, src_idx, zeros):
    mesh = plsc.VectorSubcoreMesh(core_axis_name="c", subcore_axis_name="s")

    @functools.partial(
        pl.kernel,
        out_type=jax.ShapeDtypeStruct((4 * N_PAD, 8), jnp.float32),
        mesh=mesh,
        compiler_params=pltpu.CompilerParams(use_tc_tiling_on_sc=False, needs_layout_passes=False),
        scratch_types=[
            pltpu.VMEM((3 * CHUNK,), jnp.float32),
            pltpu.VMEM((CHUNK, 8), jnp.float32),
            pltpu.VMEM((CHUNK,), jnp.int32),
            pltpu.VMEM((NG, G), jnp.int32),
            pltpu.VMEM((DRN, 8), jnp.float32),
            pltpu.VMEM_SHARED((N_PAD, 8), jnp.float32),
            pltpu.SemaphoreType.DMA,
        ],
    )
    def k(disp_hbm, di_hbm, si_hbm, z_hbm, out_hbm,
          fbuf, dbuf, ibuf, sbuf, zbuf, acc, ssem):
        cid = lax.axis_index("c")
        sid = lax.axis_index("s")
        wid = sid * NC + cid
        row0 = sid * PT
        base = wid * EW

        lanes = lax.iota(jnp.int32, 16)
        rowoff, coloff = [], []
        for v in range(3):
            pvec = lanes + (v * 16)
            q = lax.shift_right_logical(pvec * 21846, 16)
            rowoff.append(q)
            coloff.append(pvec - q * 3)

        pltpu.sync_copy(z_hbm, zbuf)
        pltpu.sync_copy(z_hbm, dbuf.at[pl.ds(0, DRN)])
        pltpu.sync_copy(z_hbm, dbuf.at[pl.ds(36, DRN)])

        for p, idx_hbm in enumerate([di_hbm, si_hbm]):
            for j in range(PT // DRN):
                pltpu.sync_copy(zbuf, acc.at[pl.ds(row0 + j * DRN, DRN)])
            plsc.subcore_barrier()

            def body(i, carry, idx_hbm=idx_hbm):
                pltpu.sync_copy(
                    disp_hbm.at[pl.ds(3 * (base + i * CHUNK), 3 * CHUNK)], fbuf)
                pltpu.sync_copy(idx_hbm.at[pl.ds(base + i * CHUNK, CHUNK)], ibuf)

                def repack(b, c2):
                    eb = b * 16
                    for v in range(3):
                        vals = fbuf[pl.ds(b * 48 + v * 16, 16)]
                        plsc.store_scatter(
                            dbuf, [rowoff[v] + eb, coloff[v]], vals)
                    return c2

                lax.fori_loop(0, NB, repack, 0)

                def icopy(t, c2):
                    r = lax.shift_right_logical(t * 52429, 18)
                    cb = (t - r * 5) * 16
                    sbuf[r, pl.ds(cb, 16)] = ibuf[pl.ds(t * 16, 16)]
                    return c2

                lax.fori_loop(0, CHUNK // 16, icopy, 0)

                descs = [
                    pltpu.async_copy(dbuf.at[pl.ds(j * G, G)],
                                     acc.at[sbuf.at[j]], ssem, add=True)
                    for j in range(NG)
                ]
                for d in descs:
                    d.wait()
                return carry

            lax.fori_loop(0, NCHUNK, body, 0)
            plsc.subcore_barrier()

            for j in range(PT // DRN):
                r = row0 + j * DRN
                pltpu.sync_copy(acc.at[pl.ds(r, DRN)], zbuf)
                pltpu.sync_copy(
                    zbuf, out_hbm.at[pl.ds((2 * p + cid) * N_PAD + r, DRN)])
            plsc.subcore_barrier()
            pltpu.sync_copy(z_hbm, zbuf)

    return k(disp_flat, dst_idx, src_idx, zeros)


def kernel(disp, atom_node, edge_index):
    del atom_node
    disp_flat = disp.reshape(-1)
    idx = edge_index.astype(jnp.int32)
    si = idx[0]
    di = idx[1]
    zeros = jnp.zeros((DRN, 8), jnp.float32)
    out = _sc_scatter(disp_flat, di, si, zeros).reshape(2, 2, N_PAD, 8)
    res = (out[0, 0] + out[0, 1]) - (out[1, 0] + out[1, 1])
    return res[:N_NODES, :3]


# speedup vs baseline: 5.6871x; 5.6871x over previous
import functools

import jax
import jax.numpy as jnp
from jax import lax
from jax.experimental import pallas as pl
from jax.experimental.pallas import tpu as pltpu
from jax.experimental.pallas import tpu_sc as plsc

N_NODES = 100000
N_EDGES = 6400000

NC = 2
NS = 16
NW = NC * NS
EW = N_EDGES // NW
G = 80
CHUNK = 1600
NG = CHUNK // G
NCHUNK = EW // CHUNK
NB = CHUNK // 16
PT = 6256
N_PAD = PT * NS
DRN = PT // 4


def _sc_scatter(dx, dy, dz, dst_idx, src_idx, zeros):
    mesh = plsc.VectorSubcoreMesh(core_axis_name="c", subcore_axis_name="s")

    @functools.partial(
        pl.kernel,
        out_type=jax.ShapeDtypeStruct((4 * N_PAD, 8), jnp.float32),
        mesh=mesh,
        compiler_params=pltpu.CompilerParams(use_tc_tiling_on_sc=False, needs_layout_passes=False),
        scratch_types=[
            pltpu.VMEM((CHUNK,), jnp.float32),
            pltpu.VMEM((CHUNK,), jnp.float32),
            pltpu.VMEM((CHUNK,), jnp.float32),
            pltpu.VMEM((CHUNK, 8), jnp.float32),
            pltpu.VMEM((CHUNK,), jnp.int32),
            pltpu.VMEM((NG, G), jnp.int32),
            pltpu.VMEM((DRN, 8), jnp.float32),
            pltpu.VMEM_SHARED((N_PAD, 8), jnp.float32),
            pltpu.SemaphoreType.DMA,
        ],
    )
    def k(dx_hbm, dy_hbm, dz_hbm, di_hbm, si_hbm, z_hbm, out_hbm,
          fbx, fby, fbz, dbuf, ibuf, sbuf, zbuf, acc, ssem):
        cid = lax.axis_index("c")
        sid = lax.axis_index("s")
        wid = sid * NC + cid
        row0 = sid * PT
        base = wid * EW

        lanes = lax.iota(jnp.int32, 16)
        cols = [lanes * 0 + c for c in range(3)]

        pltpu.sync_copy(z_hbm, zbuf)
        pltpu.sync_copy(z_hbm, dbuf.at[pl.ds(0, DRN)])
        pltpu.sync_copy(z_hbm, dbuf.at[pl.ds(36, DRN)])

        for p, idx_hbm in enumerate([di_hbm, si_hbm]):
            for j in range(PT // DRN):
                pltpu.sync_copy(zbuf, acc.at[pl.ds(row0 + j * DRN, DRN)])
            plsc.subcore_barrier()

            def body(i, carry, idx_hbm=idx_hbm):
                off = base + i * CHUNK
                pltpu.sync_copy(dx_hbm.at[pl.ds(off, CHUNK)], fbx)
                pltpu.sync_copy(dy_hbm.at[pl.ds(off, CHUNK)], fby)
                pltpu.sync_copy(dz_hbm.at[pl.ds(off, CHUNK)], fbz)
                pltpu.sync_copy(idx_hbm.at[pl.ds(off, CHUNK)], ibuf)

                def repack(b, c2):
                    eb = b * 16
                    rows = lanes + eb
                    for c, fb in enumerate([fbx, fby, fbz]):
                        vals = fb[pl.ds(eb, 16)]
                        plsc.store_scatter(dbuf, [rows, cols[c]], vals)
                    return c2

                lax.fori_loop(0, NB, repack, 0)

                def icopy(t, c2):
                    r = lax.shift_right_logical(t * 52429, 18)
                    cb = (t - r * 5) * 16
                    sbuf[r, pl.ds(cb, 16)] = ibuf[pl.ds(t * 16, 16)]
                    return c2

                lax.fori_loop(0, CHUNK // 16, icopy, 0)

                descs = [
                    pltpu.async_copy(dbuf.at[pl.ds(j * G, G)],
                                     acc.at[sbuf.at[j]], ssem, add=True)
                    for j in range(NG)
                ]
                for d in descs:
                    d.wait()
                return carry

            lax.fori_loop(0, NCHUNK, body, 0)
            plsc.subcore_barrier()

            for j in range(PT // DRN):
                r = row0 + j * DRN
                pltpu.sync_copy(acc.at[pl.ds(r, DRN)], zbuf)
                pltpu.sync_copy(
                    zbuf, out_hbm.at[pl.ds((2 * p + cid) * N_PAD + r, DRN)])
            plsc.subcore_barrier()
            pltpu.sync_copy(z_hbm, zbuf)

    return k(dx, dy, dz, dst_idx, src_idx, zeros)


def kernel(disp, atom_node, edge_index):
    del atom_node
    one = lax.optimization_barrier(jnp.ones((), jnp.float32))
    dx = disp[:, 0] * one
    dy = disp[:, 1] * one
    dz = disp[:, 2] * one
    idx = edge_index.astype(jnp.int32)
    si = idx[0]
    di = idx[1]
    zeros = jnp.zeros((DRN, 8), jnp.float32)
    out = _sc_scatter(dx, dy, dz, di, si, zeros).reshape(2, 2, N_PAD, 8)
    res = (out[0, 0] + out[0, 1]) - (out[1, 0] + out[1, 1])
    return res[:N_NODES, :3]


# speedup vs baseline: 9.3156x; 1.6380x over previous
import functools

import jax
import jax.numpy as jnp
from jax import lax
from jax.experimental import pallas as pl
from jax.experimental.pallas import tpu as pltpu
from jax.experimental.pallas import tpu_sc as plsc

N_NODES = 100000
N_EDGES = 6400000

NC = 2
NS = 16
NW = NC * NS
EW = N_EDGES // NW
G = 80
CHUNK = 800
NG = CHUNK // G
NCHUNK = EW // CHUNK
NB = CHUNK // 16
PT = 6256
N_PAD = PT * NS
DRN = PT // 8


def _sc_scatter(dx, dy, dz, dst_idx, src_idx, zeros):
    mesh = plsc.VectorSubcoreMesh(core_axis_name="c", subcore_axis_name="s")

    @functools.partial(
        pl.kernel,
        out_type=jax.ShapeDtypeStruct((4 * N_PAD, 8), jnp.float32),
        mesh=mesh,
        compiler_params=pltpu.CompilerParams(use_tc_tiling_on_sc=False,
                                             needs_layout_passes=False),
        scratch_types=[
            [pltpu.VMEM((CHUNK,), jnp.float32)] * 3,
            [pltpu.VMEM((CHUNK,), jnp.float32)] * 3,
            [pltpu.VMEM((NG, G), jnp.int32)] * 2,
            [pltpu.VMEM((CHUNK, 8), jnp.float32)] * 2,
            pltpu.VMEM((DRN, 8), jnp.float32),
            pltpu.VMEM_SHARED((N_PAD, 8), jnp.float32),
            [pltpu.SemaphoreType.DMA] * 2,
            pltpu.SemaphoreType.DMA,
        ],
    )
    def k(dx_hbm, dy_hbm, dz_hbm, di_hbm, si_hbm, z_hbm, out_hbm,
          fb0, fb1, sbufs, dbufs, zbuf, acc, isems, ssem):
        cid = lax.axis_index("c")
        sid = lax.axis_index("s")
        wid = sid * NC + cid
        row0 = sid * PT
        base = wid * EW
        fbs = [fb0, fb1]
        planes = [dx_hbm, dy_hbm, dz_hbm]

        lanes = lax.iota(jnp.int32, 16)
        cols = [lanes * 0 + c for c in range(3)]

        def start_in(chunk, b, idx_hbm):
            off = base + chunk * CHUNK
            for c in range(3):
                pltpu.async_copy(planes[c].at[pl.ds(off, CHUNK)],
                                 fbs[b][c], isems[b])
            for j in range(NG):
                pltpu.async_copy(idx_hbm.at[pl.ds(off + j * G, G)],
                                 sbufs[b].at[j], isems[b])

        def wait_in(b):
            for c in range(3):
                pltpu.make_async_copy(planes[c].at[pl.ds(0, CHUNK)],
                                      fbs[b][c], isems[b]).wait()
            for j in range(NG):
                pltpu.make_async_copy(di_hbm.at[pl.ds(0, G)],
                                      sbufs[b].at[j], isems[b]).wait()

        pltpu.sync_copy(z_hbm, zbuf)
        for b in range(2):
            pltpu.sync_copy(z_hbm, dbufs[b].at[pl.ds(0, DRN)])
            pltpu.sync_copy(z_hbm, dbufs[b].at[pl.ds(CHUNK - DRN, DRN)])

        for p, idx_hbm in enumerate([di_hbm, si_hbm]):
            for j in range(PT // DRN):
                pltpu.sync_copy(zbuf, acc.at[pl.ds(row0 + j * DRN, DRN)])
            plsc.subcore_barrier()

            start_in(0, 0, idx_hbm)

            def pair(t, carry, idx_hbm=idx_hbm):
                for b in range(2):
                    chunk = 2 * t + b
                    wait_in(b)

                    @pl.when(chunk + 1 < NCHUNK)
                    def _():
                        start_in(chunk + 1, 1 - b, idx_hbm)

                    def repack(blk, c2):
                        eb = blk * 16
                        rows = lanes + eb
                        for c in range(3):
                            plsc.store_scatter(dbufs[b], [rows, cols[c]],
                                               fbs[b][c][pl.ds(eb, 16)])
                        return c2

                    lax.fori_loop(0, NB, repack, 0)

                    descs = [
                        pltpu.async_copy(dbufs[b].at[pl.ds(j * G, G)],
                                         acc.at[sbufs[b].at[j]], ssem,
                                         add=True)
                        for j in range(NG)
                    ]
                    for d in descs:
                        d.wait()
                return carry

            lax.fori_loop(0, NCHUNK // 2, pair, 0)
            plsc.subcore_barrier()

            for j in range(PT // DRN):
                r = row0 + j * DRN
                pltpu.sync_copy(acc.at[pl.ds(r, DRN)], zbuf)
                pltpu.sync_copy(
                    zbuf, out_hbm.at[pl.ds((2 * p + cid) * N_PAD + r, DRN)])
            plsc.subcore_barrier()
            pltpu.sync_copy(z_hbm, zbuf)

    return k(dx, dy, dz, dst_idx, src_idx, zeros)


def kernel(disp, atom_node, edge_index):
    del atom_node
    one = lax.optimization_barrier(jnp.ones((), jnp.float32))
    dx = disp[:, 0] * one
    dy = disp[:, 1] * one
    dz = disp[:, 2] * one
    idx = edge_index.astype(jnp.int32)
    si = idx[0]
    di = idx[1]
    zeros = jnp.zeros((DRN, 8), jnp.float32)
    out = _sc_scatter(dx, dy, dz, di, si, zeros).reshape(2, 2, N_PAD, 8)
    res = (out[0, 0] + out[0, 1]) - (out[1, 0] + out[1, 1])
    return res[:N_NODES, :3]
